# baseline (speedup 1.0000x reference)
import functools

import jax
import jax.numpy as jnp
from jax.experimental import pallas as pl
from jax.experimental.pallas import tpu as pltpu

T, H, E, I, TOPK = 32, 768, 8, 768, 2
ALPHA, LIMIT, EPS = 1.702, 7.0, 1e-5
G = 2

_CONTRACT_LAST = (((1,), (1,)), ((), ()))
_CONTRACT_STD = (((1,), (0,)), ((), ()))


def _moe_kernel(x_ref, ns_ref, gw_ref, gb_ref, w1_ref, b1_ref, w2_ref,
                b2_ref, out_ref, tt_s, wt_s, wtt_s, acct_s, ht_s, b1t_s):
    e = pl.program_id(0)

    @pl.when(e == 0)
    def _setup():
        xf = x_ref[...].astype(jnp.float32)
        ms = jnp.mean(xf * xf, axis=1, keepdims=True)
        t = (xf * jax.lax.rsqrt(ms + EPS) * ns_ref[...]).astype(jnp.bfloat16)
        tt_s[...] = jnp.transpose(t)
        g = jax.lax.dot_general(
            t, gw_ref[...], _CONTRACT_LAST,
            preferred_element_type=jnp.float32).astype(jnp.bfloat16)
        g = g + gb_ref[...]
        gf = g.astype(jnp.float32)
        col = jax.lax.broadcasted_iota(jnp.int32, (T, E), 1)
        m1 = jnp.max(gf, axis=1, keepdims=True)
        i1 = jnp.min(jnp.where(gf == m1, col, E), axis=1, keepdims=True)
        sel1 = col == i1
        gf2 = jnp.where(sel1, -jnp.inf, gf)
        m2 = jnp.max(gf2, axis=1, keepdims=True)
        i2 = jnp.min(jnp.where(gf2 == m2, col, E), axis=1, keepdims=True)
        sel2 = col == i2
        p2 = 1.0 / (1.0 + jnp.exp(m1 - m2))
        p1 = 1.0 - p2
        wts = jnp.where(sel1, p1, 0.0) + jnp.where(sel2, p2, 0.0)
        wt_s[...] = wts
        wtt_s[...] = jnp.transpose(wts)
        b1t_s[...] = jnp.transpose(b1_ref[...].astype(jnp.float32))
        acct_s[...] = jnp.zeros_like(acct_s)

    tt = tt_s[...]
    w1cat = jnp.concatenate([w1_ref[j] for j in range(G)], axis=0)
    ht_s[...] = jax.lax.dot_general(w1cat, tt, _CONTRACT_STD,
                                    preferred_element_type=jnp.float32)
    lane_ie = jax.lax.broadcasted_iota(jnp.int32, (I, E), 1)
    eids = [G * e + j for j in range(G)]
    b1tg = b1t_s[0::2, :]
    b1tl = b1t_s[1::2, :]
    b1g = jnp.concatenate([
        jnp.sum(jnp.where(lane_ie == ej, b1tg, 0.0), axis=1, keepdims=True)
        for ej in eids], axis=0)
    b1l = jnp.concatenate([
        jnp.sum(jnp.where(lane_ie == ej, b1tl, 0.0), axis=1, keepdims=True)
        for ej in eids], axis=0)
    xg = jnp.minimum(ht_s[0::2, :] + b1g, LIMIT)
    xl = jnp.clip(ht_s[1::2, :] + b1l, -LIMIT, LIMIT)
    act = xg * jax.nn.sigmoid(ALPHA * xg) * (xl + 1.0)
    sub_et = jax.lax.broadcasted_iota(jnp.int32, (E, T), 0)
    wtt = wtt_s[...]
    acts = jnp.concatenate([
        (act[I * j:I * (j + 1), :]
         * jnp.sum(jnp.where(sub_et == ej, wtt, 0.0), axis=0, keepdims=True))
        for j, ej in enumerate(eids)], axis=0).astype(jnp.bfloat16)
    w2cat = jnp.concatenate([w2_ref[j] for j in range(G)], axis=1)
    acct_s[...] += jax.lax.dot_general(w2cat, acts, _CONTRACT_STD,
                                       preferred_element_type=jnp.float32)

    @pl.when(e == E // G - 1)
    def _finish():
        comb = jnp.transpose(acct_s[...])
        bias2 = jax.lax.dot_general(
            wt_s[...].astype(jnp.bfloat16), b2_ref[...], _CONTRACT_STD,
            preferred_element_type=jnp.float32)
        out_ref[...] = x_ref[...] + (comb + bias2).astype(jnp.bfloat16)


@functools.partial(jax.jit, static_argnames=())
def kernel(x, norm_scale, gate_w, gate_b, mlp1_weight, mlp1_bias,
           mlp2_weight, mlp2_bias):
    ns = norm_scale.reshape(1, H)
    gb = gate_b.reshape(1, E)

    full = lambda *shape: pl.BlockSpec(shape, lambda e: (0,) * len(shape))
    out = pl.pallas_call(
        _moe_kernel,
        grid=(E // G,),
        in_specs=[
            full(T, H),
            full(1, H),
            full(E, H),
            full(1, E),
            pl.BlockSpec((G, 2 * I, H), lambda e: (e, 0, 0)),
            full(E, 2 * I),
            pl.BlockSpec((G, H, I), lambda e: (e, 0, 0)),
            full(E, H),
        ],
        out_specs=pl.BlockSpec((T, H), lambda e: (0, 0)),
        out_shape=jax.ShapeDtypeStruct((T, H), jnp.bfloat16),
        scratch_shapes=[
            pltpu.VMEM((H, T), jnp.bfloat16),
            pltpu.VMEM((T, E), jnp.float32),
            pltpu.VMEM((E, T), jnp.float32),
            pltpu.VMEM((H, T), jnp.float32),
            pltpu.VMEM((G * 2 * I, T), jnp.float32),
            pltpu.VMEM((2 * I, E), jnp.float32),
        ],
        compiler_params=pltpu.CompilerParams(
            dimension_semantics=("arbitrary",),
        ),
    )(x, ns, gate_w, gb, mlp1_weight, mlp1_bias, mlp2_weight, mlp2_bias)
    return out


# speedup vs baseline: 1.0298x; 1.0298x over previous
import functools

import jax
import jax.numpy as jnp
from jax.experimental import pallas as pl
from jax.experimental.pallas import tpu as pltpu

T, H, E, I, TOPK = 32, 768, 8, 768, 2
ALPHA, LIMIT, EPS = 1.702, 7.0, 1e-5
G = 2

_CONTRACT_LAST = (((1,), (1,)), ((), ()))
_CONTRACT_STD = (((1,), (0,)), ((), ()))


def _moe_kernel(x_ref, ns_ref, gw_ref, gb_ref, w1_ref, b1_ref, w2_ref,
                b2_ref, out_ref, tt_s, wt_s, wtt_s, acct_s, ht_s, b1t_s):
    e = pl.program_id(0)

    @pl.when(e == 0)
    def _setup():
        xf = x_ref[...].astype(jnp.float32)
        ms = jnp.mean(xf * xf, axis=1, keepdims=True)
        t = (xf * jax.lax.rsqrt(ms + EPS) * ns_ref[...]).astype(jnp.bfloat16)
        tt_s[...] = jnp.transpose(t)
        g = jax.lax.dot_general(
            t, gw_ref[...], _CONTRACT_LAST,
            preferred_element_type=jnp.float32).astype(jnp.bfloat16)
        g = g + gb_ref[...]
        gf = g.astype(jnp.float32)
        col = jax.lax.broadcasted_iota(jnp.int32, (T, E), 1)
        m1 = jnp.max(gf, axis=1, keepdims=True)
        i1 = jnp.min(jnp.where(gf == m1, col, E), axis=1, keepdims=True)
        sel1 = col == i1
        gf2 = jnp.where(sel1, -jnp.inf, gf)
        m2 = jnp.max(gf2, axis=1, keepdims=True)
        i2 = jnp.min(jnp.where(gf2 == m2, col, E), axis=1, keepdims=True)
        sel2 = col == i2
        p2 = 1.0 / (1.0 + jnp.exp(m1 - m2))
        p1 = 1.0 - p2
        wts = jnp.where(sel1, p1, 0.0) + jnp.where(sel2, p2, 0.0)
        wt_s[...] = wts
        wtt_s[...] = jnp.transpose(wts)
        b1t_s[...] = jnp.transpose(b1_ref[...].astype(jnp.float32))
        acct_s[...] = jnp.zeros_like(acct_s)

    tt = tt_s[...]
    for j in range(G):
        ht_s[2 * I * j:2 * I * (j + 1), :] = jax.lax.dot_general(
            w1_ref[j], tt, _CONTRACT_STD, preferred_element_type=jnp.float32)
    lane_ie = jax.lax.broadcasted_iota(jnp.int32, (I, E), 1)
    eids = [G * e + j for j in range(G)]
    b1tg = b1t_s[0::2, :]
    b1tl = b1t_s[1::2, :]
    b1g = jnp.concatenate([
        jnp.sum(jnp.where(lane_ie == ej, b1tg, 0.0), axis=1, keepdims=True)
        for ej in eids], axis=0)
    b1l = jnp.concatenate([
        jnp.sum(jnp.where(lane_ie == ej, b1tl, 0.0), axis=1, keepdims=True)
        for ej in eids], axis=0)
    xg = jnp.minimum(ht_s[0::2, :] + b1g, LIMIT)
    xl = jnp.clip(ht_s[1::2, :] + b1l, -LIMIT, LIMIT)
    act = xg * jax.nn.sigmoid(ALPHA * xg) * (xl + 1.0)
    sub_et = jax.lax.broadcasted_iota(jnp.int32, (E, T), 0)
    wtt = wtt_s[...]
    ot = acct_s[...]
    for j, ej in enumerate(eids):
        wj = jnp.sum(jnp.where(sub_et == ej, wtt, 0.0), axis=0, keepdims=True)
        act_j = (act[I * j:I * (j + 1), :] * wj).astype(jnp.bfloat16)
        ot += jax.lax.dot_general(w2_ref[j], act_j, _CONTRACT_STD,
                                  preferred_element_type=jnp.float32)
    acct_s[...] = ot

    @pl.when(e == E // G - 1)
    def _finish():
        comb = jnp.transpose(acct_s[...])
        bias2 = jax.lax.dot_general(
            wt_s[...].astype(jnp.bfloat16), b2_ref[...], _CONTRACT_STD,
            preferred_element_type=jnp.float32)
        out_ref[...] = x_ref[...] + (comb + bias2).astype(jnp.bfloat16)


@functools.partial(jax.jit, static_argnames=())
def kernel(x, norm_scale, gate_w, gate_b, mlp1_weight, mlp1_bias,
           mlp2_weight, mlp2_bias):
    ns = norm_scale.reshape(1, H)
    gb = gate_b.reshape(1, E)

    full = lambda *shape: pl.BlockSpec(shape, lambda e: (0,) * len(shape))
    out = pl.pallas_call(
        _moe_kernel,
        grid=(E // G,),
        in_specs=[
            full(T, H),
            full(1, H),
            full(E, H),
            full(1, E),
            pl.BlockSpec((G, 2 * I, H), lambda e: (e, 0, 0)),
            full(E, 2 * I),
            pl.BlockSpec((G, H, I), lambda e: (e, 0, 0)),
            full(E, H),
        ],
        out_specs=pl.BlockSpec((T, H), lambda e: (0, 0)),
        out_shape=jax.ShapeDtypeStruct((T, H), jnp.bfloat16),
        scratch_shapes=[
            pltpu.VMEM((H, T), jnp.bfloat16),
            pltpu.VMEM((T, E), jnp.float32),
            pltpu.VMEM((E, T), jnp.float32),
            pltpu.VMEM((H, T), jnp.float32),
            pltpu.VMEM((G * 2 * I, T), jnp.float32),
            pltpu.VMEM((2 * I, E), jnp.float32),
        ],
        compiler_params=pltpu.CompilerParams(
            dimension_semantics=("arbitrary",),
        ),
    )(x, ns, gate_w, gb, mlp1_weight, mlp1_bias, mlp2_weight, mlp2_bias)
    return out


# speedup vs baseline: 1.1162x; 1.0839x over previous
import functools

import jax
import jax.numpy as jnp
from jax.experimental import pallas as pl
from jax.experimental.pallas import tpu as pltpu

T, H, E, I, TOPK = 32, 768, 8, 768, 2
ALPHA, LIMIT, EPS = 1.702, 7.0, 1e-5
G = 2

_CONTRACT_LAST = (((1,), (1,)), ((), ()))
_CONTRACT_STD = (((1,), (0,)), ((), ()))


def _moe_kernel(x_ref, ns_ref, gw_ref, gb_ref, w1_ref, b1_ref, w2_ref,
                b2_ref, out_ref, tt_s, wt_s, wtt_s, acct_s, ht_s, b1t_s):
    e = pl.program_id(0)

    @pl.when(e == 0)
    def _setup():
        xf = x_ref[...].astype(jnp.float32)
        ms = jnp.mean(xf * xf, axis=1, keepdims=True)
        t = (xf * jax.lax.rsqrt(ms + EPS) * ns_ref[...]).astype(jnp.bfloat16)
        tt_s[...] = jnp.transpose(t)
        g = jax.lax.dot_general(
            t, gw_ref[...], _CONTRACT_LAST,
            preferred_element_type=jnp.float32).astype(jnp.bfloat16)
        g = g + gb_ref[...]
        gf = g.astype(jnp.float32)
        col = jax.lax.broadcasted_iota(jnp.int32, (T, E), 1)
        m1 = jnp.max(gf, axis=1, keepdims=True)
        i1 = jnp.min(jnp.where(gf == m1, col, E), axis=1, keepdims=True)
        sel1 = col == i1
        gf2 = jnp.where(sel1, -jnp.inf, gf)
        m2 = jnp.max(gf2, axis=1, keepdims=True)
        i2 = jnp.min(jnp.where(gf2 == m2, col, E), axis=1, keepdims=True)
        sel2 = col == i2
        p2 = 1.0 / (1.0 + jnp.exp(m1 - m2))
        p1 = 1.0 - p2
        wts = jnp.where(sel1, p1, 0.0) + jnp.where(sel2, p2, 0.0)
        wt_s[...] = wts
        wtt_s[...] = jnp.transpose(wts)
        b1t_s[...] = jnp.transpose(b1_ref[...].astype(jnp.float32))
        acct_s[...] = jnp.zeros_like(acct_s)

    tt = tt_s[...]
    for j in range(G):
        ht_s[2 * I * j:2 * I * (j + 1), :] = jax.lax.dot_general(
            w1_ref[j], tt, _CONTRACT_STD, preferred_element_type=jnp.float32)
    lane_ie = jax.lax.broadcasted_iota(jnp.int32, (I, E), 1)
    eids = [G * e + j for j in range(G)]
    b1tg = b1t_s[0::2, :]
    b1tl = b1t_s[1::2, :]
    b1g = jnp.concatenate([
        jnp.sum(jnp.where(lane_ie == ej, b1tg, 0.0), axis=1, keepdims=True)
        for ej in eids], axis=0)
    b1l = jnp.concatenate([
        jnp.sum(jnp.where(lane_ie == ej, b1tl, 0.0), axis=1, keepdims=True)
        for ej in eids], axis=0)
    xg = jnp.minimum(ht_s[0::2, :] + b1g, LIMIT)
    xl = jnp.clip(ht_s[1::2, :] + b1l, -LIMIT, LIMIT)
    act = xg * jax.nn.sigmoid(ALPHA * xg) * (xl + 1.0)
    sub_et = jax.lax.broadcasted_iota(jnp.int32, (E, T), 0)
    wtt = wtt_s[...]
    wa = jnp.sum(jnp.where(sub_et == eids[0], wtt, 0.0), axis=0,
                 keepdims=True)
    wb = jnp.sum(jnp.where(sub_et == eids[1], wtt, 0.0), axis=0,
                 keepdims=True)
    act_a = (act[0:I, :] * wa).astype(jnp.bfloat16)
    act_b = (act[I:, :] * wb).astype(jnp.bfloat16)
    ot = jax.lax.dot_general(w2_ref[0], act_a, _CONTRACT_STD,
                             preferred_element_type=jnp.float32)
    ot += jax.lax.dot_general(w2_ref[1], act_b, _CONTRACT_STD,
                              preferred_element_type=jnp.float32)
    acct_s[...] += ot

    @pl.when(e == E // G - 1)
    def _finish():
        comb = jnp.transpose(acct_s[...])
        bias2 = jax.lax.dot_general(
            wt_s[...].astype(jnp.bfloat16), b2_ref[...], _CONTRACT_STD,
            preferred_element_type=jnp.float32)
        out_ref[...] = x_ref[...] + (comb + bias2).astype(jnp.bfloat16)


@functools.partial(jax.jit, static_argnames=())
def kernel(x, norm_scale, gate_w, gate_b, mlp1_weight, mlp1_bias,
           mlp2_weight, mlp2_bias):
    ns = norm_scale.reshape(1, H)
    gb = gate_b.reshape(1, E)

    full = lambda *shape: pl.BlockSpec(shape, lambda e: (0,) * len(shape))
    out = pl.pallas_call(
        _moe_kernel,
        grid=(E // G,),
        in_specs=[
            full(T, H),
            full(1, H),
            full(E, H),
            full(1, E),
            pl.BlockSpec((G, 2 * I, H), lambda e: (e, 0, 0)),
            full(E, 2 * I),
            pl.BlockSpec((G, H, I), lambda e: (e, 0, 0)),
            full(E, H),
        ],
        out_specs=pl.BlockSpec((T, H), lambda e: (0, 0)),
        out_shape=jax.ShapeDtypeStruct((T, H), jnp.bfloat16),
        scratch_shapes=[
            pltpu.VMEM((H, T), jnp.bfloat16),
            pltpu.VMEM((T, E), jnp.float32),
            pltpu.VMEM((E, T), jnp.float32),
            pltpu.VMEM((H, T), jnp.float32),
            pltpu.VMEM((G * 2 * I, T), jnp.float32),
            pltpu.VMEM((2 * I, E), jnp.float32),
        ],
        compiler_params=pltpu.CompilerParams(
            dimension_semantics=("arbitrary",),
        ),
    )(x, ns, gate_w, gb, mlp1_weight, mlp1_bias, mlp2_weight, mlp2_bias)
    return out
